# baseline (speedup 1.0000x reference)
import jax
import jax.numpy as jnp
from jax import lax
from jax.experimental import pallas as pl
from jax.experimental.pallas import tpu as pltpu

EMBED_DIM = 2048
TEXT_DIM = 768
NUM_EXPERTS = 64
TOP_K = 8
BATCH = 4
SEQ = 4096

CHUNK = 1024
NCHUNK = SEQ // CHUNK


def _gating_kernel(x_ref, t_ref, w1_ref, b1_ref, w2_ref, b2_ref,
                   w_out_ref, i_out_ref, sums_ref):
    b = pl.program_id(0)
    step = pl.program_id(1)

    partial = jnp.sum(x_ref[...], axis=1)
    h_part = jnp.dot(partial, w1_ref[:EMBED_DIM, :],
                     preferred_element_type=jnp.float32)

    @pl.when(step == 0)
    def _init():
        sums_ref[pl.ds(b, 1)] = h_part

    @pl.when(step != 0)
    def _accum():
        sums_ref[pl.ds(b, 1)] = sums_ref[pl.ds(b, 1)] + h_part

    @pl.when((b == BATCH - 1) & (step == NCHUNK - 1))
    def _finish():
        h = (sums_ref[...] * (1.0 / SEQ)
             + jnp.dot(t_ref[...], w1_ref[EMBED_DIM:, :],
                       preferred_element_type=jnp.float32)
             + b1_ref[...])
        h = jnp.maximum(h, 0.0)
        logits = (jnp.dot(h, w2_ref[...], preferred_element_type=jnp.float32)
                  + b2_ref[...])

        iota = lax.broadcasted_iota(jnp.int32, (BATCH, NUM_EXPERTS), 1)
        cur = logits
        vals = []
        idxs = []
        for _ in range(TOP_K):
            m = jnp.max(cur, axis=1, keepdims=True)
            sel = cur == m
            idx = jnp.min(jnp.where(sel, iota, NUM_EXPERTS),
                          axis=1, keepdims=True)
            vals.append(m)
            idxs.append(idx)
            cur = jnp.where(iota == idx, -jnp.inf, cur)
        top_v = jnp.concatenate(vals, axis=1)
        top_i = jnp.concatenate(idxs, axis=1)
        e = jnp.exp(top_v - top_v[:, 0:1])
        w = e / jnp.sum(e, axis=1, keepdims=True)
        w_out_ref[...] = w
        i_out_ref[...] = top_i


@jax.jit
def kernel(x, text_embedding, W1, b1, W2, b2):
    b1r = b1.reshape(1, EMBED_DIM)
    b2r = b2.reshape(1, NUM_EXPERTS)
    out = pl.pallas_call(
        _gating_kernel,
        grid=(BATCH, NCHUNK),
        in_specs=[
            pl.BlockSpec((1, CHUNK, EMBED_DIM), lambda b, i: (b, i, 0)),
            pl.BlockSpec((BATCH, TEXT_DIM), lambda b, i: (0, 0)),
            pl.BlockSpec((EMBED_DIM + TEXT_DIM, EMBED_DIM),
                         lambda b, i: (0, 0)),
            pl.BlockSpec((1, EMBED_DIM), lambda b, i: (0, 0)),
            pl.BlockSpec((EMBED_DIM, NUM_EXPERTS), lambda b, i: (0, 0)),
            pl.BlockSpec((1, NUM_EXPERTS), lambda b, i: (0, 0)),
        ],
        out_specs=[
            pl.BlockSpec((BATCH, TOP_K), lambda b, i: (0, 0)),
            pl.BlockSpec((BATCH, TOP_K), lambda b, i: (0, 0)),
        ],
        out_shape=[
            jax.ShapeDtypeStruct((BATCH, TOP_K), jnp.float32),
            jax.ShapeDtypeStruct((BATCH, TOP_K), jnp.int32),
        ],
        scratch_shapes=[pltpu.VMEM((BATCH, EMBED_DIM), jnp.float32)],
        compiler_params=pltpu.CompilerParams(
            dimension_semantics=("arbitrary", "arbitrary"),
            vmem_limit_bytes=120 * 1024 * 1024,
        ),
    )(x, text_embedding, W1, b1r, W2, b2r)
    return (out[0], out[1])


# speedup vs baseline: 1.0482x; 1.0482x over previous
import jax
import jax.numpy as jnp
from jax import lax
from jax.experimental import pallas as pl
from jax.experimental.pallas import tpu as pltpu

EMBED_DIM = 2048
TEXT_DIM = 768
NUM_EXPERTS = 64
TOP_K = 8
BATCH = 4
SEQ = 4096

CHUNK = 1024
NCHUNK = SEQ // CHUNK


def _gating_kernel(x_ref, t_ref, w1_ref, b1_ref, w2_ref, b2_ref,
                   w_out_ref, i_out_ref, sums_ref):
    b = pl.program_id(0)
    step = pl.program_id(1)

    partial = jnp.sum(x_ref[...], axis=1)

    @pl.when(step == 0)
    def _init():
        sums_ref[pl.ds(b, 1)] = partial

    @pl.when(step != 0)
    def _accum():
        sums_ref[pl.ds(b, 1)] = sums_ref[pl.ds(b, 1)] + partial

    @pl.when((b == BATCH - 1) & (step == NCHUNK - 1))
    def _finish():
        x_mean = sums_ref[...] * (1.0 / SEQ)
        h = (jnp.dot(x_mean, w1_ref[:EMBED_DIM, :],
                     preferred_element_type=jnp.float32)
             + jnp.dot(t_ref[...], w1_ref[EMBED_DIM:, :],
                       preferred_element_type=jnp.float32)
             + b1_ref[...])
        h = jnp.maximum(h, 0.0)
        logits = (jnp.dot(h, w2_ref[...], preferred_element_type=jnp.float32)
                  + b2_ref[...])

        iota = lax.broadcasted_iota(jnp.int32, (BATCH, NUM_EXPERTS), 1)
        cur = logits
        vals = []
        idxs = []
        for _ in range(TOP_K):
            m = jnp.max(cur, axis=1, keepdims=True)
            sel = cur == m
            idx = jnp.min(jnp.where(sel, iota, NUM_EXPERTS),
                          axis=1, keepdims=True)
            vals.append(m)
            idxs.append(idx)
            cur = jnp.where(iota == idx, -jnp.inf, cur)
        top_v = jnp.concatenate(vals, axis=1)
        top_i = jnp.concatenate(idxs, axis=1)
        e = jnp.exp(top_v - top_v[:, 0:1])
        w = e / jnp.sum(e, axis=1, keepdims=True)
        w_out_ref[...] = w
        i_out_ref[...] = top_i


@jax.jit
def kernel(x, text_embedding, W1, b1, W2, b2):
    b1r = b1.reshape(1, EMBED_DIM)
    b2r = b2.reshape(1, NUM_EXPERTS)
    out = pl.pallas_call(
        _gating_kernel,
        grid=(BATCH, NCHUNK),
        in_specs=[
            pl.BlockSpec((1, CHUNK, EMBED_DIM), lambda b, i: (b, i, 0)),
            pl.BlockSpec((BATCH, TEXT_DIM), lambda b, i: (0, 0)),
            pl.BlockSpec((EMBED_DIM + TEXT_DIM, EMBED_DIM),
                         lambda b, i: (0, 0)),
            pl.BlockSpec((1, EMBED_DIM), lambda b, i: (0, 0)),
            pl.BlockSpec((EMBED_DIM, NUM_EXPERTS), lambda b, i: (0, 0)),
            pl.BlockSpec((1, NUM_EXPERTS), lambda b, i: (0, 0)),
        ],
        out_specs=[
            pl.BlockSpec((BATCH, TOP_K), lambda b, i: (0, 0)),
            pl.BlockSpec((BATCH, TOP_K), lambda b, i: (0, 0)),
        ],
        out_shape=[
            jax.ShapeDtypeStruct((BATCH, TOP_K), jnp.float32),
            jax.ShapeDtypeStruct((BATCH, TOP_K), jnp.int32),
        ],
        scratch_shapes=[pltpu.VMEM((BATCH, EMBED_DIM), jnp.float32)],
        compiler_params=pltpu.CompilerParams(
            dimension_semantics=("arbitrary", "arbitrary"),
            vmem_limit_bytes=120 * 1024 * 1024,
        ),
    )(x, text_embedding, W1, b1r, W2, b2r)
    return (out[0], out[1])
